# baseline (speedup 1.0000x reference)
import functools

import jax
import jax.numpy as jnp
import numpy as np
from jax.experimental import pallas as pl
from jax.experimental.pallas import tpu as pltpu
from jax.experimental.pallas import tpu_sc as plsc

_HIDDEN = 1024
_S = 2
_BITS = 8
_R = _HIDDEN // _BITS
_M = 4
_J = _S * _R
_G = 2 * _M


def _pack_matrix():
    f = np.arange(_S * _HIDDEN)
    s, rem = f // _HIDDEN, f % _HIDDEN
    r, k = rem // _BITS, rem % _BITS
    p = np.zeros((_S * _HIDDEN, _J), np.float32)
    p[f, s * _R + r] = np.exp2(k)
    return jnp.asarray(p, jnp.bfloat16)


def _expand_matrix():
    e = np.zeros((_J, _J * _G), np.float32)
    for j in range(_J):
        e[j, j * _G:(j + 1) * _G] = 1.0
    return jnp.asarray(e, jnp.bfloat16)


def _idx_body(h_ref, w_ref, p_ref, e_ref, o_ref, pe_ref):
    h = h_ref[0]
    q = jnp.dot(h.astype(jnp.bfloat16), w_ref[...],
                preferred_element_type=jnp.float32)
    bits = (q > 0).astype(jnp.bfloat16)
    codes = jnp.dot(bits, p_ref[...],
                    preferred_element_type=jnp.float32).astype(jnp.int32)
    t = codes.shape[0]
    prev = jnp.concatenate(
        [jnp.zeros((1, _J), jnp.int32), codes[:-1, :]], axis=0)
    j16 = jax.lax.broadcasted_iota(jnp.int32, (t, _J), 1) << 16
    gidx = j16 + prev + (codes << 8)
    o_ref[0] = gidx
    par = (gidx & 1).astype(jnp.bfloat16)
    pe_ref[0] = jnp.dot(par, e_ref[...],
                        preferred_element_type=jnp.float32
                        ).astype(jnp.bfloat16)


def _out_body(h_ref, m8_ref, pe_ref, w_ref, o_ref, *, blk, t):
    i = pl.program_id(0)
    m8 = m8_ref[...]
    pe = pe_ref[...].astype(jnp.float32)
    cl = ((jax.lax.broadcasted_iota(jnp.int32, (blk, _J * _G), 1) >> 2) & 1
          ).astype(jnp.float32)
    mask = pe * cl + (1.0 - pe) * (1.0 - cl)
    rows = i * blk + jax.lax.broadcasted_iota(jnp.int32, (blk, _J * _G), 0)
    mask = jnp.where((rows % t) == 0, 0.0, mask)
    inj = jnp.dot((m8 * mask).astype(jnp.bfloat16), w_ref[...],
                  preferred_element_type=jnp.float32)
    o_ref[...] = h_ref[...] + inj


def _gather_call(bank, idx_flat, n, chunk):
    bank16 = jax.lax.bitcast_convert_type(
        bank, jnp.bfloat16).reshape(bank.shape[0] // 2, 2 * _G)
    mesh = plsc.VectorSubcoreMesh(
        core_axis_name="core", subcore_axis_name="subcore")
    n_workers = 32
    rows_w = n // n_workers
    n_chunks = rows_w // chunk

    @functools.partial(
        pl.kernel,
        out_type=jax.ShapeDtypeStruct((n, 2 * _G), jnp.bfloat16),
        mesh=mesh,
        scratch_types=[
            pltpu.VMEM((chunk,), jnp.int32),
            pltpu.VMEM((chunk,), jnp.int32),
            pltpu.VMEM((chunk,), jnp.int32),
            pltpu.VMEM((chunk,), jnp.int32),
            pltpu.VMEM((chunk, 2 * _G), jnp.bfloat16),
            pltpu.VMEM((chunk, 2 * _G), jnp.bfloat16),
            pltpu.SemaphoreType.DMA((3, 2)),
        ],
        compiler_params=pltpu.CompilerParams(use_tc_tiling_on_sc=False))
    def _gather(bank_hbm, idx_hbm, out_hbm, gi0, gi1, i0, i1, b0, b1, sems):
        gi_v, i32_v, buf_v = [gi0, gi1], [i0, i1], [b0, b1]
        cid = jax.lax.axis_index("core")
        sid = jax.lax.axis_index("subcore")
        base = (sid * 2 + cid) * rows_w

        def idx_cp(k, s):
            return pltpu.make_async_copy(
                idx_hbm.at[pl.ds(base + k * chunk, chunk)],
                gi_v[s], sems.at[0, s])

        def gat_cp(s):
            return pltpu.make_async_copy(
                bank_hbm.at[i32_v[s]], buf_v[s], sems.at[1, s])

        def out_cp(k, s):
            return pltpu.make_async_copy(
                buf_v[s], out_hbm.at[pl.ds(base + k * chunk, chunk)],
                sems.at[2, s])

        def shift_pass(s):
            @pl.loop(0, chunk // 16)
            def _(i):
                i32_v[s][pl.ds(i * 16, 16)] = gi_v[s][pl.ds(i * 16, 16)] >> 1

        for s in range(2):
            c = idx_cp(s, s)
            c.start()
            c.wait()
            shift_pass(s)
            gat_cp(s).start()
        for j in range(n_chunks):
            s = j % 2
            gat_cp(s).wait()
            oc = out_cp(j, s)
            oc.start()
            jn = j + 2
            if jn < n_chunks:
                c = idx_cp(jn, s)
                c.start()
                c.wait()
                shift_pass(s)
                oc.wait()
                gat_cp(s).start()
            else:
                oc.wait()

    out16 = _gather(bank16, idx_flat)
    return jax.lax.bitcast_convert_type(
        out16.reshape(n, _G, 2), jnp.float32)


def kernel(hidden_states, latent_q_weight, bank, out_proj):
    b, t, h = hidden_states.shape
    fused_t = latent_q_weight.reshape(_S * h, h).T.astype(jnp.bfloat16)
    pmat = _pack_matrix()
    emat = _expand_matrix()
    w8 = jnp.broadcast_to(
        out_proj.T.astype(jnp.bfloat16).reshape(_J, 1, _M, h),
        (_J, 2, _M, h)).reshape(_J * _G, h)

    gidx, pexp = pl.pallas_call(
        _idx_body,
        grid=(b,),
        in_specs=[
            pl.BlockSpec((1, t, h), lambda i: (i, 0, 0)),
            pl.BlockSpec((h, _S * h), lambda i: (0, 0)),
            pl.BlockSpec((_S * h, _J), lambda i: (0, 0)),
            pl.BlockSpec((_J, _J * _G), lambda i: (0, 0)),
        ],
        out_specs=[
            pl.BlockSpec((1, t, _J), lambda i: (i, 0, 0)),
            pl.BlockSpec((1, t, _J * _G), lambda i: (i, 0, 0)),
        ],
        out_shape=[
            jax.ShapeDtypeStruct((b, t, _J), jnp.int32),
            jax.ShapeDtypeStruct((b, t, _J * _G), jnp.bfloat16),
        ],
    )(hidden_states, fused_t, pmat, emat)

    n = b * t * _J
    mem8 = _gather_call(bank, gidx.reshape(n), n, 2048)
    mem8_flat = mem8.reshape(b * t, _J * _G)
    pexp_flat = pexp.reshape(b * t, _J * _G)

    blk = 512
    hidden_flat = hidden_states.reshape(b * t, h)
    out = pl.pallas_call(
        functools.partial(_out_body, blk=blk, t=t),
        grid=(b * t // blk,),
        in_specs=[
            pl.BlockSpec((blk, h), lambda i: (i, 0)),
            pl.BlockSpec((blk, _J * _G), lambda i: (i, 0)),
            pl.BlockSpec((blk, _J * _G), lambda i: (i, 0)),
            pl.BlockSpec((_J * _G, h), lambda i: (0, 0)),
        ],
        out_specs=pl.BlockSpec((blk, h), lambda i: (i, 0)),
        out_shape=jax.ShapeDtypeStruct((b * t, h), jnp.float32),
    )(hidden_flat, mem8_flat, pexp_flat, w8)
    return out.reshape(b, t, h)


# speedup vs baseline: 1.8343x; 1.8343x over previous
import functools

import jax
import jax.numpy as jnp
import numpy as np
from jax.experimental import pallas as pl
from jax.experimental.pallas import tpu as pltpu
from jax.experimental.pallas import tpu_sc as plsc

_HIDDEN = 1024
_S = 2
_BITS = 8
_R = _HIDDEN // _BITS
_M = 4
_J = _S * _R
_G = 2 * _M


def _pack_matrix():
    f = np.arange(_S * _HIDDEN)
    s, rem = f // _HIDDEN, f % _HIDDEN
    r, k = rem // _BITS, rem % _BITS
    p = np.zeros((_S * _HIDDEN, _J), np.float32)
    p[f, s * _R + r] = np.exp2(k)
    return jnp.asarray(p, jnp.bfloat16)


def _expand_matrix():
    e = np.zeros((_J, _J * _G), np.float32)
    for j in range(_J):
        e[j, j * _G:(j + 1) * _G] = 1.0
    return jnp.asarray(e, jnp.bfloat16)


def _idx_body(h_ref, w_ref, p_ref, e_ref, o_ref, pe_ref):
    h = h_ref[0]
    q = jnp.dot(h.astype(jnp.bfloat16), w_ref[...],
                preferred_element_type=jnp.float32)
    bits = (q > 0).astype(jnp.bfloat16)
    codes = jnp.dot(bits, p_ref[...],
                    preferred_element_type=jnp.float32).astype(jnp.int32)
    t = codes.shape[0]
    prev = jnp.concatenate(
        [jnp.zeros((1, _J), jnp.int32), codes[:-1, :]], axis=0)
    j16 = jax.lax.broadcasted_iota(jnp.int32, (t, _J), 1) << 16
    gidx = j16 + prev + (codes << 8)
    o_ref[0] = gidx
    par = (gidx & 1).astype(jnp.bfloat16)
    pe_ref[0] = jnp.dot(par, e_ref[...],
                        preferred_element_type=jnp.float32
                        ).astype(jnp.bfloat16)


def _out_body(h_ref, m8_ref, pe_ref, w_ref, o_ref, *, blk, t):
    i = pl.program_id(0)
    m8 = m8_ref[...]
    pe = pe_ref[...].astype(jnp.float32)
    cl = ((jax.lax.broadcasted_iota(jnp.int32, (blk, _J * _G), 1) >> 2) & 1
          ).astype(jnp.float32)
    mask = pe * cl + (1.0 - pe) * (1.0 - cl)
    rows = i * blk + jax.lax.broadcasted_iota(jnp.int32, (blk, _J * _G), 0)
    mask = jnp.where((rows % t) == 0, 0.0, mask)
    inj = jnp.dot((m8 * mask).astype(jnp.bfloat16), w_ref[...],
                  preferred_element_type=jnp.float32)
    o_ref[...] = h_ref[...] + inj


def _gather_call(bank, idx_flat, n, chunk):
    bank16 = jax.lax.bitcast_convert_type(
        bank, jnp.bfloat16).reshape(bank.shape[0], _G)
    n2 = 2 * n
    mesh = plsc.VectorSubcoreMesh(
        core_axis_name="core", subcore_axis_name="subcore")
    n_workers = 32
    rows_w = n2 // n_workers
    n_chunks = rows_w // chunk

    @functools.partial(
        pl.kernel,
        out_type=jax.ShapeDtypeStruct((n2, _G), jnp.bfloat16),
        mesh=mesh,
        scratch_types=[
            pltpu.VMEM((chunk,), jnp.int32),
            pltpu.VMEM((chunk,), jnp.int32),
            pltpu.VMEM((chunk, _G), jnp.bfloat16),
            pltpu.VMEM((chunk, _G), jnp.bfloat16),
            pltpu.SemaphoreType.DMA((3, 2)),
        ],
        compiler_params=pltpu.CompilerParams(use_tc_tiling_on_sc=False))
    def _gather(bank_hbm, idx_hbm, out_hbm, gi0, gi1, b0, b1, sems):
        gi_v, buf_v = [gi0, gi1], [b0, b1]
        cid = jax.lax.axis_index("core")
        sid = jax.lax.axis_index("subcore")
        base = (sid * 2 + cid) * rows_w

        def idx_cp(k, s):
            return pltpu.make_async_copy(
                idx_hbm.at[pl.ds(base + k * chunk, chunk)],
                gi_v[s], sems.at[0, s])

        def gat_cp(s):
            return pltpu.make_async_copy(
                bank_hbm.at[gi_v[s]], buf_v[s], sems.at[1, s])

        def out_cp(k, s):
            return pltpu.make_async_copy(
                buf_v[s], out_hbm.at[pl.ds(base + k * chunk, chunk)],
                sems.at[2, s])

        for s in range(2):
            c = idx_cp(s, s)
            c.start()
            c.wait()
            gat_cp(s).start()
        for j in range(n_chunks):
            s = j % 2
            gat_cp(s).wait()
            oc = out_cp(j, s)
            oc.start()
            jn = j + 2
            if jn < n_chunks:
                c = idx_cp(jn, s)
                c.start()
                c.wait()
                oc.wait()
                gat_cp(s).start()
            else:
                oc.wait()

    out16 = _gather(bank16, idx_flat)
    return jax.lax.bitcast_convert_type(
        out16.reshape(n, _G, 2), jnp.float32)


def kernel(hidden_states, latent_q_weight, bank, out_proj):
    b, t, h = hidden_states.shape
    fused_t = latent_q_weight.reshape(_S * h, h).T.astype(jnp.bfloat16)
    pmat = _pack_matrix()
    emat = _expand_matrix()
    w8 = jnp.broadcast_to(
        out_proj.T.astype(jnp.bfloat16).reshape(_J, 1, _M, h),
        (_J, 2, _M, h)).reshape(_J * _G, h)

    gidx, pexp = pl.pallas_call(
        _idx_body,
        grid=(b,),
        in_specs=[
            pl.BlockSpec((1, t, h), lambda i: (i, 0, 0)),
            pl.BlockSpec((h, _S * h), lambda i: (0, 0)),
            pl.BlockSpec((_S * h, _J), lambda i: (0, 0)),
            pl.BlockSpec((_J, _J * _G), lambda i: (0, 0)),
        ],
        out_specs=[
            pl.BlockSpec((1, t, _J), lambda i: (i, 0, 0)),
            pl.BlockSpec((1, t, _J * _G), lambda i: (i, 0, 0)),
        ],
        out_shape=[
            jax.ShapeDtypeStruct((b, t, _J), jnp.int32),
            jax.ShapeDtypeStruct((b, t, _J * _G), jnp.bfloat16),
        ],
    )(hidden_states, fused_t, pmat, emat)

    n = b * t * _J
    idxp = ((gidx.reshape(n, 1) & ~1)
            + jnp.arange(2, dtype=jnp.int32)).reshape(2 * n)
    mem8 = _gather_call(bank, idxp, n, 2048)
    mem8_flat = mem8.reshape(b * t, _J * _G)
    pexp_flat = pexp.reshape(b * t, _J * _G)

    blk = 512
    hidden_flat = hidden_states.reshape(b * t, h)
    out = pl.pallas_call(
        functools.partial(_out_body, blk=blk, t=t),
        grid=(b * t // blk,),
        in_specs=[
            pl.BlockSpec((blk, h), lambda i: (i, 0)),
            pl.BlockSpec((blk, _J * _G), lambda i: (i, 0)),
            pl.BlockSpec((blk, _J * _G), lambda i: (i, 0)),
            pl.BlockSpec((_J * _G, h), lambda i: (0, 0)),
        ],
        out_specs=pl.BlockSpec((blk, h), lambda i: (i, 0)),
        out_shape=jax.ShapeDtypeStruct((b * t, h), jnp.float32),
    )(hidden_flat, mem8_flat, pexp_flat, w8)
    return out.reshape(b, t, h)


# speedup vs baseline: 4.0348x; 2.1997x over previous
import functools

import jax
import jax.numpy as jnp
import numpy as np
from jax.experimental import pallas as pl
from jax.experimental.pallas import tpu as pltpu
from jax.experimental.pallas import tpu_sc as plsc

_HIDDEN = 1024
_S = 2
_BITS = 8
_R = _HIDDEN // _BITS
_M = 4
_J = _S * _R


def _pack_matrix():
    f = np.arange(_S * _HIDDEN)
    s, rem = f // _HIDDEN, f % _HIDDEN
    r, k = rem // _BITS, rem % _BITS
    p = np.zeros((_S * _HIDDEN, _J), np.float32)
    p[f, s * _R + r] = np.exp2(k)
    return jnp.asarray(p, jnp.bfloat16)


def _idx_body(h_ref, w_ref, p_ref, o_ref):
    h = h_ref[0]
    q = jnp.dot(h.astype(jnp.bfloat16), w_ref[...],
                preferred_element_type=jnp.float32)
    bits = (q > 0).astype(jnp.bfloat16)
    codes = jnp.dot(bits, p_ref[...],
                    preferred_element_type=jnp.float32).astype(jnp.int32)
    t = codes.shape[0]
    prev = jnp.concatenate(
        [jnp.zeros((1, _J), jnp.int32), codes[:-1, :]], axis=0)
    j16 = jax.lax.broadcasted_iota(jnp.int32, (t, _J), 1) << 16
    o_ref[0] = j16 + prev + (codes << 8)


def _out_body(h_ref, m_ref, w_ref, o_ref, *, blk, t):
    i = pl.program_id(0)
    rows = i * blk + jax.lax.broadcasted_iota(jnp.int32, (blk, _HIDDEN), 0)
    m = jnp.where((rows % t) == 0, 0.0, m_ref[...])
    inj = jnp.dot(m.astype(jnp.bfloat16), w_ref[...],
                  preferred_element_type=jnp.float32)
    o_ref[...] = h_ref[...] + inj


def _gather_call(bank, gidx_flat, n, window):
    bank16 = jax.lax.bitcast_convert_type(
        bank, jnp.bfloat16).reshape(bank.shape[0], 2 * _M)
    mesh = plsc.VectorSubcoreMesh(
        core_axis_name="core", subcore_axis_name="subcore")

    @functools.partial(
        pl.kernel,
        out_type=jax.ShapeDtypeStruct((n, 2 * _M), jnp.bfloat16),
        mesh=mesh,
        compiler_params=pltpu.CompilerParams(use_tc_tiling_on_sc=False))
    def _gather(bank_hbm, idx_hbm, out_hbm):
        def body(i_vmem, o_vmem):
            pltpu.sync_copy(bank_hbm.at[i_vmem.at[0]], o_vmem)

        pltpu.emit_pipeline(
            body,
            grid=(n // window,),
            in_specs=[pl.BlockSpec((1, window), lambda i: (0, i))],
            out_specs=[pl.BlockSpec((window, 2 * _M), lambda i: (i, 0))],
            core_axis_name=("core", "subcore"),
            dimension_semantics=(pltpu.PARALLEL,),
        )(idx_hbm, out_hbm)

    out16 = _gather(bank16, gidx_flat)
    return jax.lax.bitcast_convert_type(
        out16.reshape(n, _M, 2), jnp.float32)


def kernel(hidden_states, latent_q_weight, bank, out_proj):
    b, t, h = hidden_states.shape
    fused_t = latent_q_weight.reshape(_S * h, h).T.astype(jnp.bfloat16)
    pmat = _pack_matrix()
    out_proj_t = out_proj.T.astype(jnp.bfloat16)

    gidx = pl.pallas_call(
        _idx_body,
        grid=(b,),
        in_specs=[
            pl.BlockSpec((1, t, h), lambda i: (i, 0, 0)),
            pl.BlockSpec((h, _S * h), lambda i: (0, 0)),
            pl.BlockSpec((_S * h, _J), lambda i: (0, 0)),
        ],
        out_specs=pl.BlockSpec((1, t, _J), lambda i: (i, 0, 0)),
        out_shape=jax.ShapeDtypeStruct((b, t, _J), jnp.int32),
    )(hidden_states, fused_t, pmat)

    n = b * t * _J
    mem = _gather_call(bank, gidx.reshape(1, n), n, 128)
    mem_flat = mem.reshape(b * t, _J * _M)

    blk = 512
    hidden_flat = hidden_states.reshape(b * t, h)
    out = pl.pallas_call(
        functools.partial(_out_body, blk=blk, t=t),
        grid=(b * t // blk,),
        in_specs=[
            pl.BlockSpec((blk, h), lambda i: (i, 0)),
            pl.BlockSpec((blk, h), lambda i: (i, 0)),
            pl.BlockSpec((h, h), lambda i: (0, 0)),
        ],
        out_specs=pl.BlockSpec((blk, h), lambda i: (i, 0)),
        out_shape=jax.ShapeDtypeStruct((b * t, h), jnp.float32),
    )(hidden_flat, mem_flat, out_proj_t)
    return out.reshape(b, t, h)
